# baseline (speedup 1.0000x reference)
import functools

import jax
import jax.numpy as jnp
from jax import lax
from jax.experimental import pallas as pl
from jax.experimental.pallas import tpu as pltpu
from jax.experimental.pallas import tpu_sc as plsc

B = 32
N = 8192
D = 128
TOPK = 256
CH = 8192
BB = 4
KCOL = 8


def _tc_body(x_ref, w_ref, b_ref, o_ref):
    w = w_ref[...]
    for bb in range(BB):
        x = x_ref[bb]
        e = lax.dot_general(w, x, (((0,), (1,)), ((), ())),
                            preferred_element_type=jnp.float32)
        e = e[0:1, :] + b_ref[0]
        bu = lax.bitcast_convert_type(e, jnp.uint32)
        neg = bu >= jnp.uint32(0x80000000)
        fkey = jnp.where(neg, bu, ~bu & jnp.uint32(0x7FFFFFFF))
        fkey = jnp.where(bu == jnp.uint32(0x80000000), jnp.uint32(0x7FFFFFFF),
                         fkey)
        o_ref[bb] = lax.bitcast_convert_type(fkey, jnp.int32)


_tc_keys = pl.pallas_call(
    _tc_body,
    grid=(B // BB,),
    in_specs=[
        pl.BlockSpec((BB, CH, D), lambda i: (i, 0, 0)),
        pl.BlockSpec((D, KCOL), lambda i: (0, 0)),
        pl.BlockSpec(memory_space=pltpu.SMEM),
    ],
    out_specs=pl.BlockSpec((BB, 1, CH), lambda i: (i, 0, 0)),
    out_shape=jax.ShapeDtypeStruct((B, 1, N), jnp.int32),
)


def _byte(k_i32, shift):
    ku = plsc.bitcast(k_i32, jnp.uint32)
    return ((ku >> jnp.uint32(shift)) & jnp.uint32(0xFF)).astype(jnp.int32)


def _sc_body(keys_hbm, flat_hbm, out_hbm,
             keys_v, ck_v, ci_v, dk_v, di_v, hist_v, hist2_v, gidx_v,
             rows_v, sem):
    bidx = lax.axis_index("s") * 2 + lax.axis_index("c")
    pltpu.sync_copy(keys_hbm.at[bidx], keys_v)

    iota = lax.iota(jnp.int32, 16)
    lane_base = iota * 257
    zeros16 = jnp.zeros((16,), jnp.int32)
    ones16 = jnp.ones((16,), jnp.int32)

    def zero_hist(i, c):
        hist_v[pl.ds(i * 16, 16)] = zeros16
        return c

    def hist_scan(g0):
        T = jnp.int32(0)
        gnew = jnp.int32(0)
        found = jnp.int32(0)
        carry = jnp.int32(0)
        for v in range(16):
            tot = zeros16
            for l in range(16):
                tot = tot + hist_v[pl.ds(l * 257 + v * 16, 16)]
            cum = plsc.cumsum(tot) + carry
            cond = (g0 + cum) >= TOPK
            firstl = jnp.min(jnp.where(cond, iota, jnp.int32(999)))
            hit = (firstl < 999).astype(jnp.int32)
            cum_at = jnp.sum(jnp.where(iota == firstl, cum, 0))
            tot_at = jnp.sum(jnp.where(iota == firstl, tot, 0))
            take = (hit == 1) & (found == 0)
            T = jnp.where(take, v * 16 + firstl, T)
            gnew = jnp.where(take, g0 + cum_at - tot_at, gnew)
            found = jnp.maximum(found, hit)
            carry = carry + jnp.sum(tot)
        return T, gnew

    lax.fori_loop(0, 257, zero_hist, 0)

    def cnt1(i, c):
        for u in range(4):
            k = keys_v[pl.ds(i * 64 + u * 16, 16)]
            d = _byte(k, 24)
            plsc.addupdate_scatter(hist_v, [lane_base + d], ones16)
        return c

    lax.fori_loop(0, N // 64, cnt1, 0)
    t1, g1 = hist_scan(jnp.int32(0))

    lax.fori_loop(0, 257, zero_hist, 0)

    def cnt2(i, c):
        for u in range(4):
            k = keys_v[pl.ds(i * 64 + u * 16, 16)]
            d1 = _byte(k, 24)
            d2 = _byte(k, 16)
            plsc.addupdate_scatter(hist_v, [lane_base + d2], ones16,
                                   mask=d1 == t1)
        return c

    lax.fori_loop(0, N // 64, cnt2, 0)
    t2, _ = hist_scan(g1)
    pref_t = t1 * 256 + t2

    def compact(i, base):
        for u in range(4):
            k = keys_v[pl.ds(i * 64 + u * 16, 16)]
            ku = plsc.bitcast(k, jnp.uint32)
            pref = (ku >> jnp.uint32(16)).astype(jnp.int32)
            m = pref <= pref_t
            mi = jnp.where(m, 1, 0)
            cs = plsc.cumsum(mi)
            pos = base + cs - mi
            plsc.store_scatter(ck_v, [pos], k, mask=m)
            plsc.store_scatter(ci_v, [pos], i * 64 + u * 16 + iota, mask=m)
            base = base + jnp.sum(mi)
        return base

    S = lax.fori_loop(0, N // 64, compact, jnp.int32(0))
    nv = (S + 15) // 16

    bufs = [(ck_v, ci_v, dk_v, di_v), (dk_v, di_v, ck_v, ci_v)]
    for p in range(4):
        sk, sv, tk, tv = bufs[p % 2]
        for i in range(16):
            hist2_v[pl.ds(i * 16, 16)] = zeros16

        def count(i, c, sk=sk, shift=8 * p):
            valid = (i * 16 + iota) < S
            d = _byte(sk[pl.ds(i * 16, 16)], shift)
            occ, lastm = plsc.scan_count(d, mask=valid)
            plsc.addupdate_scatter(hist2_v, [d], occ, mask=lastm)
            return c

        lax.fori_loop(0, nv, count, 0)

        carry = jnp.int32(0)
        for v in range(16):
            vals = hist2_v[pl.ds(v * 16, 16)]
            cumv = plsc.cumsum(vals)
            hist2_v[pl.ds(v * 16, 16)] = cumv - vals + carry
            carry = carry + jnp.sum(vals)

        def perm(i, c, sk=sk, sv=sv, tk=tk, tv=tv, shift=8 * p):
            valid = (i * 16 + iota) < S
            k = sk[pl.ds(i * 16, 16)]
            vv = sv[pl.ds(i * 16, 16)]
            d = _byte(k, shift)
            occ, lastm = plsc.scan_count(d, mask=valid)
            basev = plsc.load_gather(hist2_v, [d])
            pos = basev + occ - 1
            plsc.store_scatter(tk, [pos], k, mask=valid)
            plsc.store_scatter(tv, [pos], vv, mask=valid)
            plsc.addupdate_scatter(hist2_v, [d], occ, mask=lastm)
            return c

        lax.fori_loop(0, nv, perm, 0)

    for i in range(TOPK // 16):
        v = ci_v[pl.ds(i * 16, 16)] + bidx * N
        v = jnp.clip(v, 0, B * N - 1)
        gidx_v[i // 8, pl.ds((i % 8) * 16, 16)] = v

    cp0 = pltpu.async_copy(flat_hbm.at[gidx_v.at[0]], rows_v.at[pl.ds(0, 128)], sem)
    cp1 = pltpu.async_copy(flat_hbm.at[gidx_v.at[1]], rows_v.at[pl.ds(128, 128)], sem)
    cp0.wait()
    cp1.wait()
    pltpu.sync_copy(rows_v, out_hbm.at[bidx])


_sc_topk_gather = functools.partial(
    pl.kernel,
    out_type=jax.ShapeDtypeStruct((B, TOPK, D), jnp.float32),
    mesh=plsc.VectorSubcoreMesh(core_axis_name="c", subcore_axis_name="s",
                                num_cores=2, num_subcores=16),
    compiler_params=pltpu.CompilerParams(needs_layout_passes=False),
    scratch_types=[
        pltpu.VMEM((N,), jnp.int32),
        pltpu.VMEM((N,), jnp.int32),
        pltpu.VMEM((N,), jnp.int32),
        pltpu.VMEM((N,), jnp.int32),
        pltpu.VMEM((N,), jnp.int32),
        pltpu.VMEM((16 * 257,), jnp.int32),
        pltpu.VMEM((256,), jnp.int32),
        pltpu.VMEM((2, 128), jnp.int32),
        pltpu.VMEM((TOPK, D), jnp.float32),
        pltpu.SemaphoreType.DMA,
    ],
)(_sc_body)


def kernel(inputs, W, b):
    w_pad = jnp.zeros((D, KCOL), jnp.float32).at[:, 0].set(W[0])
    fkeys = _tc_keys(inputs, w_pad, b)[:, 0, :]
    flat = inputs.reshape(B * N, D)
    return _sc_topk_gather(fkeys, flat)


# speedup vs baseline: 1.0441x; 1.0441x over previous
import functools

import jax
import jax.numpy as jnp
from jax import lax
from jax.experimental import pallas as pl
from jax.experimental.pallas import tpu as pltpu
from jax.experimental.pallas import tpu_sc as plsc

B = 32
N = 8192
D = 128
TOPK = 256
CH = 8192
BB = 2
KCOL = 8


def _tc_body(x_ref, w_ref, b_ref, o_ref):
    w = w_ref[...]
    for bb in range(BB):
        x = x_ref[bb]
        e = lax.dot_general(w, x, (((0,), (1,)), ((), ())),
                            preferred_element_type=jnp.float32)
        e = e[0:1, :] + b_ref[0]
        bu = lax.bitcast_convert_type(e, jnp.uint32)
        neg = bu >= jnp.uint32(0x80000000)
        fkey = jnp.where(neg, bu, ~bu & jnp.uint32(0x7FFFFFFF))
        fkey = jnp.where(bu == jnp.uint32(0x80000000), jnp.uint32(0x7FFFFFFF),
                         fkey)
        o_ref[bb] = lax.bitcast_convert_type(fkey, jnp.int32)


_tc_keys = pl.pallas_call(
    _tc_body,
    grid=(B // BB,),
    in_specs=[
        pl.BlockSpec((BB, CH, D), lambda i: (i, 0, 0)),
        pl.BlockSpec((D, KCOL), lambda i: (0, 0)),
        pl.BlockSpec(memory_space=pltpu.SMEM),
    ],
    out_specs=pl.BlockSpec((BB, 1, CH), lambda i: (i, 0, 0)),
    out_shape=jax.ShapeDtypeStruct((B, 1, N), jnp.int32),
)


def _byte(k_i32, shift):
    ku = plsc.bitcast(k_i32, jnp.uint32)
    return ((ku >> jnp.uint32(shift)) & jnp.uint32(0xFF)).astype(jnp.int32)


def _sc_body(keys_hbm, flat_hbm, out_hbm,
             keys_v, ck_v, ci_v, dk_v, di_v, hist_v, hist2_v, gidx_v,
             rows_v, sem):
    bidx = lax.axis_index("s") * 2 + lax.axis_index("c")
    pltpu.sync_copy(keys_hbm.at[bidx], keys_v)

    iota = lax.iota(jnp.int32, 16)
    lane_base = iota * 257
    zeros16 = jnp.zeros((16,), jnp.int32)
    ones16 = jnp.ones((16,), jnp.int32)

    def zero_hist(i, c):
        hist_v[pl.ds(i * 16, 16)] = zeros16
        return c

    def hist_scan(g0):
        T = jnp.int32(0)
        gnew = jnp.int32(0)
        found = jnp.int32(0)
        carry = jnp.int32(0)
        for v in range(16):
            tot = zeros16
            for l in range(16):
                tot = tot + hist_v[pl.ds(l * 257 + v * 16, 16)]
            cum = plsc.cumsum(tot) + carry
            cond = (g0 + cum) >= TOPK
            firstl = jnp.min(jnp.where(cond, iota, jnp.int32(999)))
            hit = (firstl < 999).astype(jnp.int32)
            cum_at = jnp.sum(jnp.where(iota == firstl, cum, 0))
            tot_at = jnp.sum(jnp.where(iota == firstl, tot, 0))
            take = (hit == 1) & (found == 0)
            T = jnp.where(take, v * 16 + firstl, T)
            gnew = jnp.where(take, g0 + cum_at - tot_at, gnew)
            found = jnp.maximum(found, hit)
            carry = carry + jnp.sum(tot)
        return T, gnew

    lax.fori_loop(0, 257, zero_hist, 0)

    def cnt1(i, c):
        for u in range(4):
            k = keys_v[pl.ds(i * 64 + u * 16, 16)]
            d = _byte(k, 24)
            plsc.addupdate_scatter(hist_v, [lane_base + d], ones16)
        return c

    lax.fori_loop(0, N // 64, cnt1, 0)
    t1, g1 = hist_scan(jnp.int32(0))

    lax.fori_loop(0, 257, zero_hist, 0)

    def cnt2(i, carry):
        baseA, baseB = carry
        for u in range(4):
            k = keys_v[pl.ds(i * 64 + u * 16, 16)]
            idxv = i * 64 + u * 16 + iota
            d1 = _byte(k, 24)
            d2 = _byte(k, 16)
            mB = d1 == t1
            plsc.addupdate_scatter(hist_v, [lane_base + d2], ones16, mask=mB)
            mA = d1 < t1
            miA = jnp.where(mA, 1, 0)
            csA = plsc.cumsum(miA)
            posA = baseA + csA - miA
            plsc.store_scatter(ck_v, [posA], k, mask=mA)
            plsc.store_scatter(ci_v, [posA], idxv, mask=mA)
            baseA = baseA + jnp.sum(miA)
            miB = jnp.where(mB, 1, 0)
            csB = plsc.cumsum(miB)
            posB = baseB + csB - miB
            plsc.store_scatter(dk_v, [posB], k, mask=mB)
            plsc.store_scatter(di_v, [posB], idxv, mask=mB)
            baseB = baseB + jnp.sum(miB)
        return baseA, baseB

    g1cnt, c1 = lax.fori_loop(0, N // 64, cnt2, (jnp.int32(0), jnp.int32(0)))
    t2, _ = hist_scan(g1)

    def bfilter(i, base):
        valid = (i * 16 + iota) < c1
        k = dk_v[pl.ds(i * 16, 16)]
        idxv = di_v[pl.ds(i * 16, 16)]
        m = (_byte(k, 16) <= t2) & valid
        mi = jnp.where(m, 1, 0)
        cs = plsc.cumsum(mi)
        pos = base + cs - mi
        plsc.store_scatter(ck_v, [pos], k, mask=m)
        plsc.store_scatter(ci_v, [pos], idxv, mask=m)
        return base + jnp.sum(mi)

    S = lax.fori_loop(0, (c1 + 15) // 16, bfilter, g1cnt)
    nv = (S + 15) // 16

    bufs = [(ck_v, ci_v, dk_v, di_v), (dk_v, di_v, ck_v, ci_v)]
    for p in range(4):
        sk, sv, tk, tv = bufs[p % 2]
        for i in range(16):
            hist2_v[pl.ds(i * 16, 16)] = zeros16

        def count(i, c, sk=sk, shift=8 * p):
            valid = (i * 16 + iota) < S
            d = _byte(sk[pl.ds(i * 16, 16)], shift)
            occ, lastm = plsc.scan_count(d, mask=valid)
            plsc.addupdate_scatter(hist2_v, [d], occ, mask=lastm)
            return c

        lax.fori_loop(0, nv, count, 0)

        carry = jnp.int32(0)
        for v in range(16):
            vals = hist2_v[pl.ds(v * 16, 16)]
            cumv = plsc.cumsum(vals)
            hist2_v[pl.ds(v * 16, 16)] = cumv - vals + carry
            carry = carry + jnp.sum(vals)

        def perm(i, c, sk=sk, sv=sv, tk=tk, tv=tv, shift=8 * p):
            valid = (i * 16 + iota) < S
            k = sk[pl.ds(i * 16, 16)]
            vv = sv[pl.ds(i * 16, 16)]
            d = _byte(k, shift)
            occ, lastm = plsc.scan_count(d, mask=valid)
            basev = plsc.load_gather(hist2_v, [d])
            pos = basev + occ - 1
            plsc.store_scatter(tk, [pos], k, mask=valid)
            plsc.store_scatter(tv, [pos], vv, mask=valid)
            plsc.addupdate_scatter(hist2_v, [d], occ, mask=lastm)
            return c

        lax.fori_loop(0, nv, perm, 0)

    for i in range(TOPK // 16):
        v = ci_v[pl.ds(i * 16, 16)] + bidx * N
        v = jnp.clip(v, 0, B * N - 1)
        gidx_v[i // 8, pl.ds((i % 8) * 16, 16)] = v

    cp0 = pltpu.async_copy(flat_hbm.at[gidx_v.at[0]], rows_v.at[pl.ds(0, 128)], sem)
    cp1 = pltpu.async_copy(flat_hbm.at[gidx_v.at[1]], rows_v.at[pl.ds(128, 128)], sem)
    cp0.wait()
    cp1.wait()
    pltpu.sync_copy(rows_v, out_hbm.at[bidx])


_sc_topk_gather = functools.partial(
    pl.kernel,
    out_type=jax.ShapeDtypeStruct((B, TOPK, D), jnp.float32),
    mesh=plsc.VectorSubcoreMesh(core_axis_name="c", subcore_axis_name="s",
                                num_cores=2, num_subcores=16),
    compiler_params=pltpu.CompilerParams(needs_layout_passes=False),
    scratch_types=[
        pltpu.VMEM((N,), jnp.int32),
        pltpu.VMEM((N,), jnp.int32),
        pltpu.VMEM((N,), jnp.int32),
        pltpu.VMEM((N,), jnp.int32),
        pltpu.VMEM((N,), jnp.int32),
        pltpu.VMEM((16 * 257,), jnp.int32),
        pltpu.VMEM((256,), jnp.int32),
        pltpu.VMEM((2, 128), jnp.int32),
        pltpu.VMEM((TOPK, D), jnp.float32),
        pltpu.SemaphoreType.DMA,
    ],
)(_sc_body)


def kernel(inputs, W, b):
    w_pad = jnp.zeros((D, KCOL), jnp.float32).at[:, 0].set(W[0])
    fkeys = _tc_keys(inputs, w_pad, b)[:, 0, :]
    flat = inputs.reshape(B * N, D)
    return _sc_topk_gather(fkeys, flat)


# speedup vs baseline: 1.1719x; 1.1224x over previous
import functools

import jax
import jax.numpy as jnp
from jax import lax
from jax.experimental import pallas as pl
from jax.experimental.pallas import tpu as pltpu
from jax.experimental.pallas import tpu_sc as plsc

B = 32
N = 8192
D = 128
TOPK = 256
CH = 8192
BB = 2
KCOL = 8


def _tc_body(x_ref, w_ref, b_ref, o_ref):
    w = w_ref[...]
    for bb in range(BB):
        x = x_ref[bb]
        e = lax.dot_general(w, x, (((0,), (1,)), ((), ())),
                            preferred_element_type=jnp.float32)
        e = e[0:1, :] + b_ref[0]
        bu = lax.bitcast_convert_type(e, jnp.uint32)
        neg = bu >= jnp.uint32(0x80000000)
        fkey = jnp.where(neg, bu, ~bu & jnp.uint32(0x7FFFFFFF))
        fkey = jnp.where(bu == jnp.uint32(0x80000000), jnp.uint32(0x7FFFFFFF),
                         fkey)
        o_ref[bb] = lax.bitcast_convert_type(fkey, jnp.int32)


_tc_keys = pl.pallas_call(
    _tc_body,
    grid=(B // BB,),
    in_specs=[
        pl.BlockSpec((BB, CH, D), lambda i: (i, 0, 0)),
        pl.BlockSpec((D, KCOL), lambda i: (0, 0)),
        pl.BlockSpec(memory_space=pltpu.SMEM),
    ],
    out_specs=pl.BlockSpec((BB, 1, CH), lambda i: (i, 0, 0)),
    out_shape=jax.ShapeDtypeStruct((B, 1, N), jnp.int32),
)


def _byte(k_i32, shift):
    ku = plsc.bitcast(k_i32, jnp.uint32)
    return ((ku >> jnp.uint32(shift)) & jnp.uint32(0xFF)).astype(jnp.int32)


def _sc_body(keys_hbm, flat_hbm, out_hbm,
             keys_v, ck_v, ci_v, dk_v, di_v, hist_v, hist2_v, gidx_v,
             rows_v, sem):
    bidx = lax.axis_index("s") * 2 + lax.axis_index("c")
    pltpu.sync_copy(keys_hbm.at[bidx], keys_v)

    iota = lax.iota(jnp.int32, 16)
    lane_base = iota * 257
    zeros16 = jnp.zeros((16,), jnp.int32)
    ones16 = jnp.ones((16,), jnp.int32)

    def zero_hist(i):
        hist_v[pl.ds(i * 16, 16)] = zeros16

    def hist_scan(g0):
        T = jnp.int32(0)
        gnew = jnp.int32(0)
        found = jnp.int32(0)
        carry = jnp.int32(0)
        for v in range(16):
            tot = zeros16
            for l in range(16):
                tot = tot + hist_v[pl.ds(l * 257 + v * 16, 16)]
            cum = plsc.cumsum(tot) + carry
            cond = (g0 + cum) >= TOPK
            firstl = jnp.min(jnp.where(cond, iota, jnp.int32(999)))
            hit = (firstl < 999).astype(jnp.int32)
            cum_at = jnp.sum(jnp.where(iota == firstl, cum, 0))
            tot_at = jnp.sum(jnp.where(iota == firstl, tot, 0))
            take = (hit == 1) & (found == 0)
            T = jnp.where(take, v * 16 + firstl, T)
            gnew = jnp.where(take, g0 + cum_at - tot_at, gnew)
            found = jnp.maximum(found, hit)
            carry = carry + jnp.sum(tot)
        return T, gnew

    plsc.parallel_loop(0, 257, unroll=4)(zero_hist)

    def cnt1(i):
        k = keys_v[pl.ds(i * 16, 16)]
        d = _byte(k, 24)
        plsc.addupdate_scatter(hist_v, [lane_base + d], ones16)

    plsc.parallel_loop(0, N // 16, unroll=8)(cnt1)
    t1, g1 = hist_scan(jnp.int32(0))

    plsc.parallel_loop(0, 257, unroll=4)(zero_hist)

    def cnt2(i, carry):
        baseA, baseB = carry
        k = keys_v[pl.ds(i * 16, 16)]
        idxv = i * 16 + iota
        d1 = _byte(k, 24)
        d2 = _byte(k, 16)
        mB = d1 == t1
        plsc.addupdate_scatter(hist_v, [lane_base + d2], ones16, mask=mB)
        mA = d1 < t1
        miA = jnp.where(mA, 1, 0)
        csA = plsc.cumsum(miA)
        posA = baseA + csA - miA
        plsc.store_scatter(ck_v, [posA], k, mask=mA)
        plsc.store_scatter(ci_v, [posA], idxv, mask=mA)
        baseA = baseA + jnp.sum(miA)
        miB = jnp.where(mB, 1, 0)
        csB = plsc.cumsum(miB)
        posB = baseB + csB - miB
        plsc.store_scatter(dk_v, [posB], k, mask=mB)
        plsc.store_scatter(di_v, [posB], idxv, mask=mB)
        baseB = baseB + jnp.sum(miB)
        return baseA, baseB

    g1cnt, c1 = plsc.parallel_loop(
        0, N // 16, unroll=4,
        carry=(jnp.int32(0), jnp.int32(0)))(cnt2)
    t2, _ = hist_scan(g1)

    def bfilter(i, base):
        valid = (i * 16 + iota) < c1
        k = dk_v[pl.ds(i * 16, 16)]
        idxv = di_v[pl.ds(i * 16, 16)]
        m = (_byte(k, 16) <= t2) & valid
        mi = jnp.where(m, 1, 0)
        cs = plsc.cumsum(mi)
        pos = base + cs - mi
        plsc.store_scatter(ck_v, [pos], k, mask=m)
        plsc.store_scatter(ci_v, [pos], idxv, mask=m)
        return base + jnp.sum(mi)

    S = lax.fori_loop(0, (c1 + 15) // 16, bfilter, g1cnt)
    nv = (S + 15) // 16

    bufs = [(ck_v, ci_v, dk_v, di_v), (dk_v, di_v, ck_v, ci_v)]
    for p in range(4):
        sk, sv, tk, tv = bufs[p % 2]
        for i in range(16):
            hist2_v[pl.ds(i * 16, 16)] = zeros16

        def count(i, c, sk=sk, shift=8 * p):
            valid = (i * 16 + iota) < S
            d = _byte(sk[pl.ds(i * 16, 16)], shift)
            occ, lastm = plsc.scan_count(d, mask=valid)
            plsc.addupdate_scatter(hist2_v, [d], occ, mask=lastm)
            return c

        lax.fori_loop(0, nv, count, 0)

        carry = jnp.int32(0)
        for v in range(16):
            vals = hist2_v[pl.ds(v * 16, 16)]
            cumv = plsc.cumsum(vals)
            hist2_v[pl.ds(v * 16, 16)] = cumv - vals + carry
            carry = carry + jnp.sum(vals)

        def perm(i, c, sk=sk, sv=sv, tk=tk, tv=tv, shift=8 * p):
            valid = (i * 16 + iota) < S
            k = sk[pl.ds(i * 16, 16)]
            vv = sv[pl.ds(i * 16, 16)]
            d = _byte(k, shift)
            occ, lastm = plsc.scan_count(d, mask=valid)
            basev = plsc.load_gather(hist2_v, [d])
            pos = basev + occ - 1
            plsc.store_scatter(tk, [pos], k, mask=valid)
            plsc.store_scatter(tv, [pos], vv, mask=valid)
            plsc.addupdate_scatter(hist2_v, [d], occ, mask=lastm)
            return c

        lax.fori_loop(0, nv, perm, 0)

    for i in range(TOPK // 16):
        v = ci_v[pl.ds(i * 16, 16)] + bidx * N
        v = jnp.clip(v, 0, B * N - 1)
        gidx_v[i // 8, pl.ds((i % 8) * 16, 16)] = v

    cp0 = pltpu.async_copy(flat_hbm.at[gidx_v.at[0]], rows_v.at[pl.ds(0, 128)], sem)
    cp1 = pltpu.async_copy(flat_hbm.at[gidx_v.at[1]], rows_v.at[pl.ds(128, 128)], sem)
    cp0.wait()
    cp1.wait()
    pltpu.sync_copy(rows_v, out_hbm.at[bidx])


_sc_topk_gather = functools.partial(
    pl.kernel,
    out_type=jax.ShapeDtypeStruct((B, TOPK, D), jnp.float32),
    mesh=plsc.VectorSubcoreMesh(core_axis_name="c", subcore_axis_name="s",
                                num_cores=2, num_subcores=16),
    compiler_params=pltpu.CompilerParams(needs_layout_passes=False),
    scratch_types=[
        pltpu.VMEM((N,), jnp.int32),
        pltpu.VMEM((N,), jnp.int32),
        pltpu.VMEM((N,), jnp.int32),
        pltpu.VMEM((N,), jnp.int32),
        pltpu.VMEM((N,), jnp.int32),
        pltpu.VMEM((16 * 257,), jnp.int32),
        pltpu.VMEM((256,), jnp.int32),
        pltpu.VMEM((2, 128), jnp.int32),
        pltpu.VMEM((TOPK, D), jnp.float32),
        pltpu.SemaphoreType.DMA,
    ],
)(_sc_body)


def kernel(inputs, W, b):
    w_pad = jnp.zeros((D, KCOL), jnp.float32).at[:, 0].set(W[0])
    fkeys = _tc_keys(inputs, w_pad, b)[:, 0, :]
    flat = inputs.reshape(B * N, D)
    return _sc_topk_gather(fkeys, flat)
